# baseline (speedup 1.0000x reference)
import functools

import jax
import jax.numpy as jnp
from jax import lax
from jax.experimental import pallas as pl
from jax.experimental.pallas import tpu as pltpu
from jax.experimental.pallas import tpu_sc as plsc

B = 16384
D = 64
L = 16
NC = 2
NS = 16
NW = NC * NS
BPW = B // NW
CH = 128
NCHUNK = BPW // CH

_BITREV4 = [0, 8, 4, 12, 2, 10, 6, 14, 1, 9, 5, 13, 3, 11, 7, 15]

_DNUMS = lax.GatherDimensionNumbers(
    offset_dims=(), collapsed_slice_dims=(0,), start_index_map=(0,))


def _shuffle(p, perm):
    return lax.gather(p, perm, _DNUMS, (1,),
                      mode=lax.GatherScatterMode.PROMISE_IN_BOUNDS)


def _gmf_body(user_hbm, item_hbm, uemb_hbm, iemb_hbm, w_hbm, bias_hbm,
              out_hbm,
              uidx_v, iidx_v, ubuf0, ubuf1, ibuf0, ibuf1, w_v, bias_v, out_v,
              su0, su1, si0, si1):
    wid = lax.axis_index("s") * NC + lax.axis_index("c")
    base = wid * BPW

    pltpu.sync_copy(user_hbm.at[pl.ds(base, BPW)], uidx_v)
    pltpu.sync_copy(item_hbm.at[pl.ds(base, BPW)], iidx_v)
    pltpu.sync_copy(w_hbm, w_v)
    pltpu.sync_copy(bias_hbm, bias_v)

    ubufs = (ubuf0, ubuf1)
    ibufs = (ibuf0, ibuf1)
    usems = (su0, su1)
    isems = (si0, si1)

    def fire(c):
        p = c % 2
        cu = pltpu.async_copy(uemb_hbm.at[uidx_v.at[pl.ds(c * CH, CH)]],
                              ubufs[p], usems[p])
        ci = pltpu.async_copy(iemb_hbm.at[iidx_v.at[pl.ds(c * CH, CH)]],
                              ibufs[p], isems[p])
        return cu, ci

    wv = tuple(w_v[pl.ds(j * L, L)] for j in range(4))
    bias = bias_v[...]
    lane = lax.broadcasted_iota(jnp.int32, (L,), 0)
    perms = tuple((lane ^ (8 >> s))[:, None] for s in range(4))
    masks = tuple((lane & (8 >> s)) == 0 for s in range(4))

    def merge(a, b, lvl):
        m = masks[lvl]
        return (jnp.where(m, a, _shuffle(b, perms[lvl]))
                + jnp.where(m, _shuffle(a, perms[lvl]), b))

    def make_group(ub, ib, cbase):
        def group(g, _):
            stack = []
            for t in range(L):
                r = _BITREV4[t]
                b = g * L + r
                v = ub[b, pl.ds(0, L)] * ib[b, pl.ds(0, L)] * wv[0]
                for j in range(1, 4):
                    v = v + ub[b, pl.ds(j * L, L)] * ib[b, pl.ds(j * L, L)] * wv[j]
                lvl = 0
                while stack and stack[-1][0] == lvl:
                    pv = stack.pop()[1]
                    v = merge(pv, v, lvl)
                    lvl += 1
                stack.append((lvl, v))
            out_v[pl.ds(cbase + g * L, L)] = stack[0][1] + bias
            return _
        return group

    handles = fire(0)
    for c in range(NCHUNK):
        nxt = fire(c + 1) if c + 1 < NCHUNK else None
        handles[0].wait()
        handles[1].wait()
        p = c % 2
        lax.fori_loop(0, CH // L, make_group(ubufs[p], ibufs[p], c * CH), 0)
        handles = nxt

    pltpu.sync_copy(out_v, out_hbm.at[pl.ds(base, BPW)])


@jax.jit
def kernel(user, item, mf_user_embed, mf_item_embed, final_w, final_b):
    w_flat = final_w.reshape(D)
    bias16 = jnp.tile(final_b.reshape(1), L)
    mesh = plsc.VectorSubcoreMesh(core_axis_name="c", subcore_axis_name="s")
    run = functools.partial(
        pl.kernel,
        mesh=mesh,
        compiler_params=pltpu.CompilerParams(use_tc_tiling_on_sc=False),
        out_type=jax.ShapeDtypeStruct((B,), jnp.float32),
        scratch_types=[
            pltpu.VMEM((BPW,), jnp.int32),
            pltpu.VMEM((BPW,), jnp.int32),
            pltpu.VMEM((CH, D), jnp.float32),
            pltpu.VMEM((CH, D), jnp.float32),
            pltpu.VMEM((CH, D), jnp.float32),
            pltpu.VMEM((CH, D), jnp.float32),
            pltpu.VMEM((D,), jnp.float32),
            pltpu.VMEM((L,), jnp.float32),
            pltpu.VMEM((BPW,), jnp.float32),
            pltpu.SemaphoreType.DMA,
            pltpu.SemaphoreType.DMA,
            pltpu.SemaphoreType.DMA,
            pltpu.SemaphoreType.DMA,
        ],
    )(_gmf_body)
    out = run(user.astype(jnp.int32), item.astype(jnp.int32),
              mf_user_embed, mf_item_embed, w_flat, bias16)
    return out.reshape(B, 1)
